# baseline (speedup 1.0000x reference)
import functools

import jax
import jax.numpy as jnp
from jax import lax
from jax.experimental import pallas as pl
from jax.experimental.pallas import tpu as pltpu
from jax.experimental.pallas import tpu_sc as plsc


def _tc_add_body(x_ref, pe_ref, o_ref):
    o_ref[...] = x_ref[...] + pe_ref[...]


def _tc_kernel(x, pe):
    B, S, D = x.shape
    BS = 2048
    grid = (S // BS, B)
    return pl.pallas_call(
        _tc_add_body,
        grid=grid,
        in_specs=[
            pl.BlockSpec((1, BS, D), lambda s, b: (b, s, 0)),
            pl.BlockSpec((BS, D), lambda s, b: (s, 0)),
        ],
        out_specs=pl.BlockSpec((1, BS, D), lambda s, b: (b, s, 0)),
        out_shape=jax.ShapeDtypeStruct((B, S, D), x.dtype),
        compiler_params=pltpu.CompilerParams(
            dimension_semantics=("arbitrary", "arbitrary"),
        ),
    )(x, pe)


_L = 16


def _sc_kernel(x, pe):
    B, S, D = x.shape
    NC, NS = 2, 16
    NW = NC * NS
    rows_w = S // NW
    CH = 32
    n_chunks = rows_w // CH
    groups = CH * D // _L

    mesh = plsc.VectorSubcoreMesh(core_axis_name="c", subcore_axis_name="s")

    @functools.partial(
        pl.kernel,
        mesh=mesh,
        out_type=jax.ShapeDtypeStruct((B, S * D), jnp.float32),
        scratch_types=[
            pltpu.VMEM((CH * D,), jnp.float32),
            pltpu.VMEM((CH * D,), jnp.float32),
        ],
    )
    def sc_add(x_hbm, pe_hbm, out_hbm, pe_v, x_v):
        wid = lax.axis_index("s") * NC + lax.axis_index("c")
        base = wid * (rows_w * D)
        for ci in range(n_chunks):
            off = base + ci * (CH * D)
            pltpu.sync_copy(pe_hbm.at[pl.ds(off, CH * D)], pe_v)
            for b in range(B):
                pltpu.sync_copy(x_hbm.at[b].at[pl.ds(off, CH * D)], x_v)

                def add_body(j, _):
                    sl = pl.ds(j * _L, _L)
                    x_v[sl] = x_v[sl] + pe_v[sl]
                    return 0

                lax.fori_loop(0, groups, add_body, 0)
                pltpu.sync_copy(x_v, out_hbm.at[b].at[pl.ds(off, CH * D)])

    out = sc_add(x.reshape(B, S * D), pe.reshape(-1)[: S * D])
    return out.reshape(B, S, D)


def kernel(x, pe):
    return _sc_kernel(x, pe)


# speedup vs baseline: 1.7285x; 1.7285x over previous
import functools

import jax
import jax.numpy as jnp
from jax import lax
from jax.experimental import pallas as pl
from jax.experimental.pallas import tpu as pltpu
from jax.experimental.pallas import tpu_sc as plsc


def _tc_add_body(x_ref, pe_ref, o_ref):
    o_ref[...] = x_ref[...] + pe_ref[...]


def _tc_kernel(x, pe):
    B, S, D = x.shape
    BS = 2048
    grid = (S // BS, B)
    return pl.pallas_call(
        _tc_add_body,
        grid=grid,
        in_specs=[
            pl.BlockSpec((1, BS, D), lambda s, b: (b, s, 0)),
            pl.BlockSpec((BS, D), lambda s, b: (s, 0)),
        ],
        out_specs=pl.BlockSpec((1, BS, D), lambda s, b: (b, s, 0)),
        out_shape=jax.ShapeDtypeStruct((B, S, D), x.dtype),
        compiler_params=pltpu.CompilerParams(
            dimension_semantics=("arbitrary", "arbitrary"),
        ),
    )(x, pe)


_L = 16


def _sc_kernel(x, pe):
    B, S, D = x.shape
    NC, NS = 2, 16
    NW = NC * NS
    rows_w = S // NW
    CH = 16
    n_chunks = rows_w // CH
    groups = CH * D // _L
    NBUF = 3
    steps = [(ci, b) for ci in range(n_chunks) for b in range(B)]

    mesh = plsc.VectorSubcoreMesh(core_axis_name="c", subcore_axis_name="s")

    @functools.partial(
        pl.kernel,
        mesh=mesh,
        out_type=jax.ShapeDtypeStruct((B, S * D), jnp.float32),
        scratch_types=(
            [pltpu.VMEM((CH * D,), jnp.float32) for _ in range(NBUF)]
            + [pltpu.VMEM((CH * D,), jnp.float32) for _ in range(2)]
            + [pltpu.SemaphoreType.DMA for _ in range(NBUF + NBUF + 2)]
        ),
    )
    def sc_add(x_hbm, pe_hbm, out_hbm, *scratch):
        x_v = scratch[:NBUF]
        pe_v = scratch[NBUF:NBUF + 2]
        in_sem = scratch[NBUF + 2:2 * NBUF + 2]
        out_sem = scratch[2 * NBUF + 2:3 * NBUF + 2]
        pe_sem = scratch[3 * NBUF + 2:]
        wid = lax.axis_index("s") * NC + lax.axis_index("c")
        base = wid * (rows_w * D)

        def chunk_sl(ci):
            return pl.ds(base + ci * (CH * D), CH * D)

        in_cp = [None] * NBUF
        out_cp = [None] * NBUF
        pe_cp = [
            pltpu.async_copy(pe_hbm.at[chunk_sl(ci)], pe_v[ci % 2], pe_sem[ci % 2])
            for ci in range(min(2, n_chunks))
        ]
        for t in range(NBUF - 1):
            ci, b = steps[t]
            in_cp[t] = pltpu.async_copy(
                x_hbm.at[b].at[chunk_sl(ci)], x_v[t], in_sem[t])

        for t, (ci, b) in enumerate(steps):
            buf = t % NBUF
            in_cp[buf].wait()
            if b == 0:
                pe_cp[ci % 2].wait()
            peb = pe_v[ci % 2]
            xb = x_v[buf]

            @plsc.parallel_loop(0, groups, unroll=8)
            def _(j):
                sl = pl.ds(j * _L, _L)
                xb[sl] = xb[sl] + peb[sl]

            out_cp[buf] = pltpu.async_copy(
                xb, out_hbm.at[b].at[chunk_sl(ci)], out_sem[buf])
            if b == B - 1 and ci + 2 < n_chunks:
                pe_cp[ci % 2] = pltpu.async_copy(
                    pe_hbm.at[chunk_sl(ci + 2)], pe_v[ci % 2], pe_sem[ci % 2])
            nt = t + NBUF - 1
            if nt < len(steps):
                nci, nb = steps[nt]
                nbuf = nt % NBUF
                if out_cp[nbuf] is not None:
                    out_cp[nbuf].wait()
                in_cp[nbuf] = pltpu.async_copy(
                    x_hbm.at[nb].at[chunk_sl(nci)], x_v[nbuf], in_sem[nbuf])
        for t in range(max(0, len(steps) - NBUF), len(steps)):
            out_cp[t % NBUF].wait()

    out = sc_add(x.reshape(B, S * D), pe.reshape(-1)[: S * D])
    return out.reshape(B, S, D)


def kernel(x, pe):
    return _sc_kernel(x, pe)


# speedup vs baseline: 1.7360x; 1.0043x over previous
import functools

import jax
import jax.numpy as jnp
from jax import lax
from jax.experimental import pallas as pl
from jax.experimental.pallas import tpu as pltpu
from jax.experimental.pallas import tpu_sc as plsc


def _tc_add_body(x_ref, pe_ref, o_ref):
    o_ref[...] = x_ref[...] + pe_ref[...]


def _tc_kernel(x, pe):
    B, S, D = x.shape
    BS = 2048
    grid = (S // BS, B)
    return pl.pallas_call(
        _tc_add_body,
        grid=grid,
        in_specs=[
            pl.BlockSpec((1, BS, D), lambda s, b: (b, s, 0)),
            pl.BlockSpec((BS, D), lambda s, b: (s, 0)),
        ],
        out_specs=pl.BlockSpec((1, BS, D), lambda s, b: (b, s, 0)),
        out_shape=jax.ShapeDtypeStruct((B, S, D), x.dtype),
        compiler_params=pltpu.CompilerParams(
            dimension_semantics=("arbitrary", "arbitrary"),
        ),
    )(x, pe)


_L = 16


def _hybrid_kernel(x, pe):
    B, S, D = x.shape
    SCB = 1
    sc_out = _sc_kernel(x, pe, b_lo=B - SCB)
    BS = 2048
    tc_out = pl.pallas_call(
        _tc_add_body,
        grid=(S // BS, B - SCB),
        in_specs=[
            pl.BlockSpec((1, BS, D), lambda s, b: (b, s, 0)),
            pl.BlockSpec((BS, D), lambda s, b: (s, 0)),
        ],
        out_specs=pl.BlockSpec((1, BS, D), lambda s, b: (b, s, 0)),
        out_shape=jax.ShapeDtypeStruct((B - SCB, S, D), x.dtype),
        compiler_params=pltpu.CompilerParams(
            dimension_semantics=("arbitrary", "arbitrary"),
        ),
    )(x, pe)
    return jnp.concatenate([tc_out, sc_out], axis=0)


def _sc_kernel(x, pe, b_lo=0):
    B, S, D = x.shape
    NC, NS = 2, 16
    NW = NC * NS
    rows_w = S // NW
    CH = 16
    n_chunks = rows_w // CH
    groups = CH * D // _L
    NBUF = 3
    nb = B - b_lo
    steps = [(ci, b) for ci in range(n_chunks) for b in range(b_lo, B)]

    mesh = plsc.VectorSubcoreMesh(core_axis_name="c", subcore_axis_name="s")

    @functools.partial(
        pl.kernel,
        mesh=mesh,
        out_type=jax.ShapeDtypeStruct((nb, S * D), jnp.float32),
        scratch_types=(
            [pltpu.VMEM((CH * D,), jnp.float32) for _ in range(NBUF)]
            + [pltpu.VMEM((CH * D,), jnp.float32) for _ in range(2)]
            + [pltpu.SemaphoreType.DMA for _ in range(NBUF + NBUF + 2)]
        ),
    )
    def sc_add(x_hbm, pe_hbm, out_hbm, *scratch):
        x_v = scratch[:NBUF]
        pe_v = scratch[NBUF:NBUF + 2]
        in_sem = scratch[NBUF + 2:2 * NBUF + 2]
        out_sem = scratch[2 * NBUF + 2:3 * NBUF + 2]
        pe_sem = scratch[3 * NBUF + 2:]
        wid = lax.axis_index("s") * NC + lax.axis_index("c")
        base = wid * (rows_w * D)

        def chunk_sl(ci):
            return pl.ds(base + ci * (CH * D), CH * D)

        in_cp = [None] * NBUF
        out_cp = [None] * NBUF
        pe_cp = [
            pltpu.async_copy(pe_hbm.at[chunk_sl(ci)], pe_v[ci % 2], pe_sem[ci % 2])
            for ci in range(min(2, n_chunks))
        ]
        for t in range(NBUF - 1):
            ci, b = steps[t]
            in_cp[t] = pltpu.async_copy(
                x_hbm.at[b].at[chunk_sl(ci)], x_v[t], in_sem[t])

        for t, (ci, b) in enumerate(steps):
            buf = t % NBUF
            in_cp[buf].wait()
            if b == b_lo:
                pe_cp[ci % 2].wait()
            peb = pe_v[ci % 2]
            xb = x_v[buf]

            @plsc.parallel_loop(0, groups, unroll=8)
            def _(j):
                sl = pl.ds(j * _L, _L)
                xb[sl] = xb[sl] + peb[sl]

            out_cp[buf] = pltpu.async_copy(
                xb, out_hbm.at[b - b_lo].at[chunk_sl(ci)], out_sem[buf])
            if b == B - 1 and ci + 2 < n_chunks:
                pe_cp[ci % 2] = pltpu.async_copy(
                    pe_hbm.at[chunk_sl(ci + 2)], pe_v[ci % 2], pe_sem[ci % 2])
            nt = t + NBUF - 1
            if nt < len(steps):
                nci, nb = steps[nt]
                nbuf = nt % NBUF
                if out_cp[nbuf] is not None:
                    out_cp[nbuf].wait()
                in_cp[nbuf] = pltpu.async_copy(
                    x_hbm.at[nb].at[chunk_sl(nci)], x_v[nbuf], in_sem[nbuf])
        for t in range(max(0, len(steps) - NBUF), len(steps)):
            out_cp[t % NBUF].wait()

    out = sc_add(x.reshape(B, S * D), pe.reshape(-1)[: S * D])
    return out.reshape(nb, S, D)


def kernel(x, pe):
    return _hybrid_kernel(x, pe)


# speedup vs baseline: 4.4917x; 2.5875x over previous
import functools

import jax
import jax.numpy as jnp
from jax import lax
from jax.experimental import pallas as pl
from jax.experimental.pallas import tpu as pltpu
from jax.experimental.pallas import tpu_sc as plsc


def _tc_add_body(x_ref, pe_ref, o_ref):
    o_ref[...] = x_ref[...] + pe_ref[...]


def _tc_kernel(x, pe):
    B, S, D = x.shape
    BS = 2048
    grid = (S // BS, B)
    return pl.pallas_call(
        _tc_add_body,
        grid=grid,
        in_specs=[
            pl.BlockSpec((1, BS, D), lambda s, b: (b, s, 0)),
            pl.BlockSpec((BS, D), lambda s, b: (s, 0)),
        ],
        out_specs=pl.BlockSpec((1, BS, D), lambda s, b: (b, s, 0)),
        out_shape=jax.ShapeDtypeStruct((B, S, D), x.dtype),
        compiler_params=pltpu.CompilerParams(
            dimension_semantics=("arbitrary", "arbitrary"),
        ),
    )(x, pe)


_L = 16


def _hybrid_kernel(x, pe):
    B, S, D = x.shape
    SCB = 1
    sc_out = _sc_kernel(x, pe, b_lo=B - SCB)
    BS = 2048
    tc_out = pl.pallas_call(
        _tc_add_body,
        grid=(S // BS, B - SCB),
        in_specs=[
            pl.BlockSpec((1, BS, D), lambda s, b: (b, s, 0)),
            pl.BlockSpec((BS, D), lambda s, b: (s, 0)),
        ],
        out_specs=pl.BlockSpec((1, BS, D), lambda s, b: (b, s, 0)),
        out_shape=jax.ShapeDtypeStruct((B - SCB, S, D), x.dtype),
        compiler_params=pltpu.CompilerParams(
            dimension_semantics=("arbitrary", "arbitrary"),
        ),
    )(x, pe)
    return jnp.concatenate([tc_out, sc_out], axis=0)


def _sc_kernel(x, pe, b_lo=0, full_out=False):
    B, S, D = x.shape
    NC, NS = 2, 16
    NW = NC * NS
    rows_w = S // NW
    CH = 16
    n_chunks = rows_w // CH
    NBUF = 3
    nb = B - b_lo
    ob_lo = b_lo if full_out else 0
    steps = [(ci, b) for ci in range(n_chunks) for b in range(b_lo, B)]

    mesh = plsc.VectorSubcoreMesh(core_axis_name="c", subcore_axis_name="s")

    @functools.partial(
        pl.kernel,
        mesh=mesh,
        out_type=jax.ShapeDtypeStruct((B if full_out else nb, S, D), jnp.float32),
        compiler_params=pltpu.CompilerParams(use_tc_tiling_on_sc=True),
        scratch_types=(
            [pltpu.VMEM((CH, D), jnp.float32) for _ in range(NBUF)]
            + [pltpu.VMEM((CH, D), jnp.float32) for _ in range(2)]
            + [pltpu.SemaphoreType.DMA for _ in range(NBUF + NBUF + 2)]
        ),
    )
    def sc_add(x_hbm, pe_hbm, out_hbm, *scratch):
        x_v = scratch[:NBUF]
        pe_v = scratch[NBUF:NBUF + 2]
        in_sem = scratch[NBUF + 2:2 * NBUF + 2]
        out_sem = scratch[2 * NBUF + 2:3 * NBUF + 2]
        pe_sem = scratch[3 * NBUF + 2:]
        wid = lax.axis_index("s") * NC + lax.axis_index("c")
        base = wid * rows_w

        def rows(ci):
            return pl.ds(base + ci * CH, CH)

        in_cp = [None] * NBUF
        out_cp = [None] * NBUF
        pe_cp = [
            pltpu.async_copy(pe_hbm.at[rows(ci)], pe_v[ci % 2], pe_sem[ci % 2])
            for ci in range(min(2, n_chunks))
        ]
        for t in range(NBUF - 1):
            ci, b = steps[t]
            in_cp[t] = pltpu.async_copy(x_hbm.at[b].at[rows(ci)], x_v[t], in_sem[t])

        for t, (ci, b) in enumerate(steps):
            buf = t % NBUF
            in_cp[buf].wait()
            if b == b_lo:
                pe_cp[ci % 2].wait()
            peb = pe_v[ci % 2]
            xb = x_v[buf]

            @plsc.parallel_loop(0, CH * D // _L, unroll=8)
            def _(j):
                r = j // (D // _L)
                sl = pl.ds((j % (D // _L)) * _L, _L)
                xb[r, sl] = xb[r, sl] + peb[r, sl]

            out_cp[buf] = pltpu.async_copy(
                xb, out_hbm.at[b - ob_lo].at[rows(ci)], out_sem[buf])
            if b == B - 1 and ci + 2 < n_chunks:
                pe_cp[ci % 2] = pltpu.async_copy(
                    pe_hbm.at[rows(ci + 2)], pe_v[ci % 2], pe_sem[ci % 2])
            nt = t + NBUF - 1
            if nt < len(steps):
                nci, nbb = steps[nt]
                nbuf = nt % NBUF
                if out_cp[nbuf] is not None:
                    out_cp[nbuf].wait()
                in_cp[nbuf] = pltpu.async_copy(
                    x_hbm.at[nbb].at[rows(nci)], x_v[nbuf], in_sem[nbuf])
        for t in range(max(0, len(steps) - NBUF), len(steps)):
            out_cp[t % NBUF].wait()

    return sc_add(x, pe)


def kernel(x, pe):
    return _sc_kernel(x, pe)


# speedup vs baseline: 4.9623x; 1.1048x over previous
import functools

import jax
import jax.numpy as jnp
from jax import lax
from jax.experimental import pallas as pl
from jax.experimental.pallas import tpu as pltpu
from jax.experimental.pallas import tpu_sc as plsc


def _tc_add_body(x_ref, pe_ref, o_ref):
    o_ref[...] = x_ref[...] + pe_ref[...]


def _tc_kernel(x, pe):
    B, S, D = x.shape
    BS = 2048
    grid = (S // BS, B)
    return pl.pallas_call(
        _tc_add_body,
        grid=grid,
        in_specs=[
            pl.BlockSpec((1, BS, D), lambda s, b: (b, s, 0)),
            pl.BlockSpec((BS, D), lambda s, b: (s, 0)),
        ],
        out_specs=pl.BlockSpec((1, BS, D), lambda s, b: (b, s, 0)),
        out_shape=jax.ShapeDtypeStruct((B, S, D), x.dtype),
        compiler_params=pltpu.CompilerParams(
            dimension_semantics=("arbitrary", "arbitrary"),
        ),
    )(x, pe)


_L = 16


def _hybrid_kernel(x, pe):
    B, S, D = x.shape
    BS = 2048
    sc_s = S // 2
    sc_full = _sc_kernel(x, pe, b_lo=B - 1, full_out=True, seq_rows=sc_s)

    def body(x_ref, pe_ref, alias_ref, o_ref):
        del alias_ref
        o_ref[...] = x_ref[...] + pe_ref[...]

    def s_of(i):
        return jnp.where(i < 3, 0, 1)

    def b_of(i):
        return jnp.where(i < 3, i, i - 3)

    return pl.pallas_call(
        body,
        grid=(2 * B - 1,),
        in_specs=[
            pl.BlockSpec((1, BS, D), lambda i: (b_of(i), s_of(i), 0)),
            pl.BlockSpec((BS, D), lambda i: (s_of(i), 0)),
            pl.BlockSpec(memory_space=pl.ANY),
        ],
        out_specs=pl.BlockSpec((1, BS, D), lambda i: (b_of(i), s_of(i), 0)),
        out_shape=jax.ShapeDtypeStruct((B, S, D), x.dtype),
        input_output_aliases={2: 0},
        compiler_params=pltpu.CompilerParams(
            dimension_semantics=("arbitrary",),
        ),
    )(x, pe, sc_full)


def _sc_kernel(x, pe, b_lo=0, full_out=False, seq_rows=None):
    B, S, D = x.shape
    NC, NS = 2, 16
    NW = NC * NS
    if seq_rows is None:
        seq_rows = S
    rows_w = seq_rows // NW
    CH = 16
    n_chunks = rows_w // CH
    NBUF = 3
    nb = B - b_lo
    ob_lo = 0 if full_out else b_lo
    steps = [(ci, b) for ci in range(n_chunks) for b in range(b_lo, B)]

    mesh = plsc.VectorSubcoreMesh(core_axis_name="c", subcore_axis_name="s")

    @functools.partial(
        pl.kernel,
        mesh=mesh,
        out_type=jax.ShapeDtypeStruct((B if full_out else nb, S, D), jnp.float32),
        compiler_params=pltpu.CompilerParams(use_tc_tiling_on_sc=True),
        scratch_types=(
            [pltpu.VMEM((CH, D), jnp.float32) for _ in range(NBUF)]
            + [pltpu.VMEM((CH, D), jnp.float32) for _ in range(2)]
            + [pltpu.SemaphoreType.DMA for _ in range(NBUF + NBUF + 2)]
        ),
    )
    def sc_add(x_hbm, pe_hbm, out_hbm, *scratch):
        x_v = scratch[:NBUF]
        pe_v = scratch[NBUF:NBUF + 2]
        in_sem = scratch[NBUF + 2:2 * NBUF + 2]
        out_sem = scratch[2 * NBUF + 2:3 * NBUF + 2]
        pe_sem = scratch[3 * NBUF + 2:]
        wid = lax.axis_index("s") * NC + lax.axis_index("c")
        base = wid * rows_w

        def rows(ci):
            return pl.ds(base + ci * CH, CH)

        in_cp = [None] * NBUF
        out_cp = [None] * NBUF
        pe_cp = [
            pltpu.async_copy(pe_hbm.at[rows(ci)], pe_v[ci % 2], pe_sem[ci % 2])
            for ci in range(min(2, n_chunks))
        ]
        for t in range(NBUF - 1):
            ci, b = steps[t]
            in_cp[t] = pltpu.async_copy(x_hbm.at[b].at[rows(ci)], x_v[t], in_sem[t])

        for t, (ci, b) in enumerate(steps):
            buf = t % NBUF
            in_cp[buf].wait()
            if b == b_lo:
                pe_cp[ci % 2].wait()
            peb = pe_v[ci % 2]
            xb = x_v[buf]

            @plsc.parallel_loop(0, CH * D // _L, unroll=8)
            def _(j):
                r = j // (D // _L)
                sl = pl.ds((j % (D // _L)) * _L, _L)
                xb[r, sl] = xb[r, sl] + peb[r, sl]

            out_cp[buf] = pltpu.async_copy(
                xb, out_hbm.at[b - ob_lo].at[rows(ci)], out_sem[buf])
            if b == B - 1 and ci + 2 < n_chunks:
                pe_cp[ci % 2] = pltpu.async_copy(
                    pe_hbm.at[rows(ci + 2)], pe_v[ci % 2], pe_sem[ci % 2])
            nt = t + NBUF - 1
            if nt < len(steps):
                nci, nbb = steps[nt]
                nbuf = nt % NBUF
                if out_cp[nbuf] is not None:
                    out_cp[nbuf].wait()
                in_cp[nbuf] = pltpu.async_copy(
                    x_hbm.at[nbb].at[rows(nci)], x_v[nbuf], in_sem[nbuf])
        for t in range(max(0, len(steps) - NBUF), len(steps)):
            out_cp[t % NBUF].wait()

    return sc_add(x, pe)


def kernel(x, pe):
    return _hybrid_kernel(x, pe)
